# baseline (speedup 1.0000x reference)
import jax
import jax.numpy as jnp
from jax import lax
from jax.experimental import pallas as pl
from jax.experimental.pallas import tpu as pltpu

_N = 32768
_D = 1024
_CHUNK = 1024
_NBUF = 8
_NCHUNK = _N // _CHUNK


def _innernode_tc_kernel(x_hbm, w_ref, b_ref, o_ref, bufs, sems):
    w = w_ref[...]
    bias = b_ref[0]

    for s in range(_NBUF):
        pltpu.make_async_copy(
            x_hbm.at[pl.ds(s * _CHUNK, _CHUNK), :], bufs.at[s], sems.at[s]
        ).start()

    for i in range(_NCHUNK):
        slot = i % _NBUF
        pltpu.make_async_copy(
            x_hbm.at[pl.ds(i * _CHUNK, _CHUNK), :], bufs.at[slot], sems.at[slot]
        ).wait()
        x = bufs[slot]
        logits = lax.dot_general(
            x, w, (((1,), (0,)), ((), ())),
            preferred_element_type=jnp.float32)
        o_ref[pl.ds(i * _CHUNK, _CHUNK)] = jnp.where(
            (logits[:, 0] + bias) > 0.0, 0, 1).astype(o_ref.dtype)
        nxt = i + _NBUF
        if nxt < _NCHUNK:
            pltpu.make_async_copy(
                x_hbm.at[pl.ds(nxt * _CHUNK, _CHUNK), :],
                bufs.at[slot], sems.at[slot]
            ).start()


def kernel(feat0, feat1, feat2, w, b):
    del feat0, feat2
    N, d = feat1.shape
    out_dtype = jnp.zeros((), dtype=jnp.int64).dtype
    w2 = w.reshape(d, 1)
    b1 = b.reshape(1)
    return pl.pallas_call(
        _innernode_tc_kernel,
        in_specs=[
            pl.BlockSpec(memory_space=pltpu.HBM),
            pl.BlockSpec(memory_space=pltpu.VMEM),
            pl.BlockSpec(memory_space=pltpu.VMEM),
        ],
        out_specs=pl.BlockSpec(memory_space=pltpu.VMEM),
        out_shape=jax.ShapeDtypeStruct((N,), out_dtype),
        scratch_shapes=[
            pltpu.VMEM((_NBUF, _CHUNK, d), jnp.float32),
            pltpu.SemaphoreType.DMA((_NBUF,)),
        ],
    )(feat1, w2, b1)


# speedup vs baseline: 1.0461x; 1.0461x over previous
import jax
import jax.numpy as jnp
from jax import lax
from jax.experimental import pallas as pl
from jax.experimental.pallas import tpu as pltpu

_N = 32768
_D = 1024
_CHUNK = 1024
_NBUF = 4
_NCHUNK = _N // _CHUNK
_NSPLIT = 4
_SUB = _CHUNK // _NSPLIT


def _innernode_tc_kernel(x_hbm, w_ref, b_ref, o_ref, bufs, sems):
    w = w_ref[...]
    bias = b_ref[0]

    def start_chunk(c, slot):
        for p in range(_NSPLIT):
            pltpu.make_async_copy(
                x_hbm.at[pl.ds(c * _CHUNK + p * _SUB, _SUB), :],
                bufs.at[slot, pl.ds(p * _SUB, _SUB)], sems.at[slot]
            ).start()

    def wait_chunk(c, slot):
        for p in range(_NSPLIT):
            pltpu.make_async_copy(
                x_hbm.at[pl.ds(c * _CHUNK + p * _SUB, _SUB), :],
                bufs.at[slot, pl.ds(p * _SUB, _SUB)], sems.at[slot]
            ).wait()

    for s in range(_NBUF):
        start_chunk(s, s)

    for i in range(_NCHUNK):
        slot = i % _NBUF
        wait_chunk(i, slot)
        x = bufs[slot]
        logits = lax.dot_general(
            x, w, (((1,), (0,)), ((), ())),
            preferred_element_type=jnp.float32)
        o_ref[pl.ds(i * _CHUNK, _CHUNK)] = jnp.where(
            (logits[:, 0] + bias) > 0.0, 0, 1).astype(o_ref.dtype)
        nxt = i + _NBUF
        if nxt < _NCHUNK:
            start_chunk(nxt, slot)


def kernel(feat0, feat1, feat2, w, b):
    del feat0, feat2
    N, d = feat1.shape
    out_dtype = jnp.zeros((), dtype=jnp.int64).dtype
    w2 = w.reshape(d, 1)
    b1 = b.reshape(1)
    return pl.pallas_call(
        _innernode_tc_kernel,
        in_specs=[
            pl.BlockSpec(memory_space=pltpu.HBM),
            pl.BlockSpec(memory_space=pltpu.VMEM),
            pl.BlockSpec(memory_space=pltpu.VMEM),
        ],
        out_specs=pl.BlockSpec(memory_space=pltpu.VMEM),
        out_shape=jax.ShapeDtypeStruct((N,), out_dtype),
        scratch_shapes=[
            pltpu.VMEM((_NBUF, _CHUNK, d), jnp.float32),
            pltpu.SemaphoreType.DMA((_NBUF,)),
        ],
    )(feat1, w2, b1)
